# baseline (speedup 1.0000x reference)
import functools

import jax
import jax.numpy as jnp
from jax import lax
from jax.experimental import pallas as pl
from jax.experimental.pallas import tpu as pltpu
from jax.experimental.pallas import tpu_sc as plsc

N = 10000
E = 160000
C_IN, H1, H2 = 256, 512, 256
EPS = 1e-5

RB = 2000
G = N // RB

F = 64
PC1 = C_IN // F
PC2 = H1 // F
EB = 128
ER = 1280
EPAD = ER * EB
RPT = ER // 16
NPAD = 10240
SPAN = NPAD // 16
WB = 128

DB = 40
DROWS = EPAD // DB
DR = DROWS // 32
DSPAN = NPAD // 16
DPAD = NPAD


@functools.lru_cache(maxsize=None)
def _make_seg(C):
    CPC = C // 2
    mesh = plsc.VectorSubcoreMesh(core_axis_name="c", subcore_axis_name="s",
                                  num_cores=2, num_subcores=16)

    @functools.partial(
        pl.kernel,
        out_type=jax.ShapeDtypeStruct((C, NPAD, F), jnp.float32),
        mesh=mesh,
        compiler_params=pltpu.CompilerParams(use_tc_tiling_on_sc=False),
        scratch_types=[
            pltpu.VMEM((RPT, EB), jnp.int32),
            pltpu.VMEM((RPT, EB), jnp.int32),
            [pltpu.VMEM((EB, F), jnp.float32) for _ in range(4)],
            pltpu.VMEM((WB, F), jnp.float32),
            pltpu.VMEM((WB, F), jnp.float32),
            pltpu.VMEM_SHARED((NPAD, F), jnp.float32),
            [pltpu.SemaphoreType.DMA for _ in range(4)],
            [pltpu.SemaphoreType.DMA for _ in range(4)],
        ],
    )
    def seg(table, src2d, dst2d, zeros_h, out, srcv, dstv, gbufs,
            zbuf, obuf, acc, gsems, ssems):
        c = lax.axis_index("c")
        s = lax.axis_index("s")
        row0 = s * RPT
        pltpu.sync_copy(src2d.at[pl.ds(row0, RPT)], srcv)
        pltpu.sync_copy(dst2d.at[pl.ds(row0, RPT)], dstv)
        pltpu.sync_copy(zeros_h, zbuf)
        for cc in range(CPC):
            chunk = c * CPC + cc
            tch = table.at[chunk]
            och = out.at[chunk]
            for i in range(SPAN // WB):
                pltpu.sync_copy(zbuf, acc.at[pl.ds(s * SPAN + i * WB, WB)])
            plsc.subcore_barrier()
            for b in range(4):
                pltpu.async_copy(tch.at[srcv.at[b]], gbufs[b], gsems[b])

            def quad(p, carry):
                j0 = p * 4
                sdescs = []
                for b in range(4):
                    pltpu.make_async_copy(tch, gbufs[b], gsems[b]).wait()
                    sdescs.append(
                        pltpu.async_copy(gbufs[b], acc.at[dstv.at[j0 + b]],
                                         ssems[b], add=True))
                for b in range(4):
                    sdescs[b].wait()

                    @pl.when(j0 + b + 4 < RPT)
                    def _():
                        pltpu.async_copy(tch.at[srcv.at[j0 + b + 4]],
                                         gbufs[b], gsems[b])
                return carry

            lax.fori_loop(0, RPT // 4, quad, 0)
            plsc.subcore_barrier()
            for i in range(SPAN // WB):
                pltpu.sync_copy(acc.at[pl.ds(s * SPAN + i * WB, WB)], obuf)
                pltpu.sync_copy(obuf, och.at[pl.ds(s * SPAN + i * WB, WB)])
    return seg


@functools.lru_cache(maxsize=None)
def _make_deg():
    mesh = plsc.VectorSubcoreMesh(core_axis_name="c", subcore_axis_name="s",
                                  num_cores=2, num_subcores=16)

    @functools.partial(
        pl.kernel,
        out_type=jax.ShapeDtypeStruct((2, DPAD), jnp.float32),
        mesh=mesh,
        compiler_params=pltpu.CompilerParams(use_tc_tiling_on_sc=False),
        scratch_types=[
            pltpu.VMEM((DR, DB), jnp.int32),
            pltpu.VMEM((DB,), jnp.float32),
            pltpu.VMEM((DSPAN,), jnp.float32),
            pltpu.VMEM((DSPAN,), jnp.float32),
            pltpu.VMEM_SHARED((DPAD,), jnp.float32),
        ],
    )
    def degk(dst4d, ones_h, zeros_h, out, dstv, onesv, zbuf, obuf, acc):
        c = lax.axis_index("c")
        s = lax.axis_index("s")
        row0 = c * (DROWS // 2) + s * DR
        pltpu.sync_copy(dst4d.at[pl.ds(row0, DR)], dstv)
        pltpu.sync_copy(ones_h, onesv)
        pltpu.sync_copy(zeros_h, zbuf)
        pltpu.sync_copy(zbuf, acc.at[pl.ds(s * DSPAN, DSPAN)])
        plsc.subcore_barrier()

        def step(j, carry):
            pltpu.sync_copy(onesv, acc.at[dstv.at[j]], add=True)
            return carry

        lax.fori_loop(0, DR, step, 0)
        plsc.subcore_barrier()
        pltpu.sync_copy(acc.at[pl.ds(s * DSPAN, DSPAN)], obuf)
        pltpu.sync_copy(obuf, out.at[c].at[pl.ds(s * DSPAN, DSPAN)])
    return degk


def _stats_call(x):
    C = x.shape[1]

    def body(x_ref, o_ref):
        i = pl.program_id(0)

        @pl.when(i == 0)
        def _():
            o_ref[...] = jnp.zeros_like(o_ref)

        xb = x_ref[...]
        o_ref[0, :] = o_ref[0, :] + jnp.sum(xb, 0)
        o_ref[1, :] = o_ref[1, :] + jnp.sum(xb * xb, 0)

    return pl.pallas_call(
        body, grid=(G,),
        in_specs=[pl.BlockSpec((RB, C), lambda i: (i, 0))],
        out_specs=pl.BlockSpec((2, C), lambda i: (0, 0)),
        out_shape=jax.ShapeDtypeStruct((2, C), jnp.float32),
    )(x)


def _prep_call(x, st, deg2, g0, b0):
    def body(x_ref, st_ref, dg_ref, g_ref, b_ref, y_ref):
        mean = st_ref[0] / N
        var = st_ref[1] / N - mean * mean
        a = g_ref[...] * lax.rsqrt(var + EPS)
        bb = b_ref[...] - mean * a
        xn = x_ref[...] * a[None, :] + bb[None, :]
        deg = dg_ref[:, 0] + dg_ref[:, 1] + 1.0
        dinv = lax.rsqrt(deg)
        y = xn * dinv[:, None]
        for c in range(PC1):
            y_ref[c] = y[:, c * F:(c + 1) * F]

    return pl.pallas_call(
        body, grid=(G,),
        in_specs=[
            pl.BlockSpec((RB, C_IN), lambda i: (i, 0)),
            pl.BlockSpec((2, C_IN), lambda i: (0, 0)),
            pl.BlockSpec((RB, 2), lambda i: (i, 0)),
            pl.BlockSpec((C_IN,), lambda i: (0,)),
            pl.BlockSpec((C_IN,), lambda i: (0,)),
        ],
        out_specs=pl.BlockSpec((PC1, RB, F), lambda i: (0, i, 0)),
        out_shape=jax.ShapeDtypeStruct((PC1, N, F), jnp.float32),
    )(x, st, deg2, g0, b0)


def _selfmm_call(y_c, deg2, W1):
    def body(y_ref, dg_ref, w_ref, r_ref):
        deg = dg_ref[:, 0] + dg_ref[:, 1] + 1.0
        dinv = lax.rsqrt(deg)
        wv = w_ref[...]
        r = jnp.zeros((RB, H1), jnp.float32)
        for c in range(PC1):
            r = r + jnp.dot(y_ref[c] * dinv[:, None], wv[c * F:(c + 1) * F],
                            preferred_element_type=jnp.float32)
        r_ref[...] = r

    return pl.pallas_call(
        body, grid=(G,),
        in_specs=[
            pl.BlockSpec((PC1, RB, F), lambda i: (0, i, 0)),
            pl.BlockSpec((RB, 2), lambda i: (i, 0)),
            pl.BlockSpec((C_IN, H1), lambda i: (0, 0)),
        ],
        out_specs=pl.BlockSpec((RB, H1), lambda i: (i, 0)),
        out_shape=jax.ShapeDtypeStruct((N, H1), jnp.float32),
    )(y_c, deg2, W1)


def _mm1_call(aggS, Ry, deg2, W1, b1):
    def body(ag_ref, r_ref, dg_ref, w_ref, b_ref, z_ref, st_ref):
        i = pl.program_id(0)
        deg = dg_ref[:, 0] + dg_ref[:, 1] + 1.0
        dinv = lax.rsqrt(deg)
        wv = w_ref[...]
        z = r_ref[...] + b_ref[...][None, :]
        for c in range(PC1):
            z = z + jnp.dot(ag_ref[c] * dinv[:, None],
                            wv[c * F:(c + 1) * F],
                            preferred_element_type=jnp.float32)
        z = jnp.maximum(z, 0.0)
        z_ref[...] = z

        @pl.when(i == 0)
        def _():
            st_ref[...] = jnp.zeros_like(st_ref)

        st_ref[0, :] = st_ref[0, :] + jnp.sum(z, 0)
        st_ref[1, :] = st_ref[1, :] + jnp.sum(z * z, 0)

    return pl.pallas_call(
        body, grid=(G,),
        in_specs=[
            pl.BlockSpec((PC1, RB, F), lambda i: (0, i, 0)),
            pl.BlockSpec((RB, H1), lambda i: (i, 0)),
            pl.BlockSpec((RB, 2), lambda i: (i, 0)),
            pl.BlockSpec((C_IN, H1), lambda i: (0, 0)),
            pl.BlockSpec((H1,), lambda i: (0,)),
        ],
        out_specs=[
            pl.BlockSpec((RB, H1), lambda i: (i, 0)),
            pl.BlockSpec((2, H1), lambda i: (0, 0)),
        ],
        out_shape=[
            jax.ShapeDtypeStruct((N, H1), jnp.float32),
            jax.ShapeDtypeStruct((2, H1), jnp.float32),
        ],
    )(aggS, Ry, deg2, W1, b1)


def _bn_call(Z, st, gamma, beta):
    def body(z_ref, st_ref, g_ref, b_ref, h_ref):
        mean = st_ref[0] / N
        var = st_ref[1] / N - mean * mean
        a = g_ref[...] * lax.rsqrt(var + EPS)
        bb = b_ref[...] - mean * a
        h = z_ref[...] * a[None, :] + bb[None, :]
        for c in range(PC2):
            h_ref[c] = h[:, c * F:(c + 1) * F]

    return pl.pallas_call(
        body, grid=(G,),
        in_specs=[
            pl.BlockSpec((RB, H1), lambda i: (i, 0)),
            pl.BlockSpec((2, H1), lambda i: (0, 0)),
            pl.BlockSpec((H1,), lambda i: (0,)),
            pl.BlockSpec((H1,), lambda i: (0,)),
        ],
        out_specs=pl.BlockSpec((PC2, RB, F), lambda i: (0, i, 0)),
        out_shape=jax.ShapeDtypeStruct((PC2, N, F), jnp.float32),
    )(Z, st, gamma, beta)


def _root_call(h_c, Wroot):
    Hout = Wroot.shape[1]

    def body(h_ref, w_ref, r_ref):
        wv = w_ref[...]
        r = jnp.dot(h_ref[0], wv[:F], preferred_element_type=jnp.float32)
        for c in range(1, PC2):
            r = r + jnp.dot(h_ref[c], wv[c * F:(c + 1) * F],
                            preferred_element_type=jnp.float32)
        r_ref[...] = r

    return pl.pallas_call(
        body, grid=(G,),
        in_specs=[
            pl.BlockSpec((PC2, RB, F), lambda i: (0, i, 0)),
            pl.BlockSpec((H1, Hout), lambda i: (0, 0)),
        ],
        out_specs=pl.BlockSpec((RB, Hout), lambda i: (i, 0)),
        out_shape=jax.ShapeDtypeStruct((N, Hout), jnp.float32),
    )(h_c, Wroot)


def _mm2_call(aggS, R, W_rel, b):
    Hout = W_rel.shape[1]

    def body(ag_ref, r_ref, w_ref, b_ref, z_ref, st_ref):
        i = pl.program_id(0)
        wv = w_ref[...]
        z = r_ref[...] + b_ref[...][None, :]
        for c in range(PC2):
            z = z + jnp.dot(ag_ref[c], wv[c * F:(c + 1) * F],
                            preferred_element_type=jnp.float32)
        z = jnp.maximum(z, 0.0)
        z_ref[...] = z

        @pl.when(i == 0)
        def _():
            st_ref[...] = jnp.zeros_like(st_ref)

        st_ref[0, :] = st_ref[0, :] + jnp.sum(z, 0)
        st_ref[1, :] = st_ref[1, :] + jnp.sum(z * z, 0)

    return pl.pallas_call(
        body, grid=(G,),
        in_specs=[
            pl.BlockSpec((PC2, RB, F), lambda i: (0, i, 0)),
            pl.BlockSpec((RB, Hout), lambda i: (i, 0)),
            pl.BlockSpec((H1, Hout), lambda i: (0, 0)),
            pl.BlockSpec((Hout,), lambda i: (0,)),
        ],
        out_specs=[
            pl.BlockSpec((RB, Hout), lambda i: (i, 0)),
            pl.BlockSpec((2, Hout), lambda i: (0, 0)),
        ],
        out_shape=[
            jax.ShapeDtypeStruct((N, Hout), jnp.float32),
            jax.ShapeDtypeStruct((2, Hout), jnp.float32),
        ],
    )(aggS, R, W_rel, b)


def _bnfinal_call(Z, st, gamma, beta):
    C = Z.shape[1]

    def body(z_ref, st_ref, g_ref, b_ref, o_ref):
        mean = st_ref[0] / N
        var = st_ref[1] / N - mean * mean
        a = g_ref[...] * lax.rsqrt(var + EPS)
        bb = b_ref[...] - mean * a
        o_ref[...] = z_ref[...] * a[None, :] + bb[None, :]

    return pl.pallas_call(
        body, grid=(G,),
        in_specs=[
            pl.BlockSpec((RB, C), lambda i: (i, 0)),
            pl.BlockSpec((2, C), lambda i: (0, 0)),
            pl.BlockSpec((C,), lambda i: (0,)),
            pl.BlockSpec((C,), lambda i: (0,)),
        ],
        out_specs=pl.BlockSpec((RB, C), lambda i: (i, 0)),
        out_shape=jax.ShapeDtypeStruct((N, C), jnp.float32),
    )(Z, st, gamma, beta)


def kernel(x, edge_index, gamma0, beta0, W1, b1, gamma1, beta1,
           W2_rel, b2, W2_root, gamma2, beta2,
           W3_rel, b3, W3_root, gamma3, beta3):
    pad = jnp.arange(EPAD - E, dtype=jnp.int32)
    srcp = jnp.concatenate([edge_index[0], pad % N])
    dstp = jnp.concatenate([edge_index[1], N + pad % (NPAD - N)])
    src2d = srcp.reshape(ER, EB)
    dst2d = dstp.reshape(ER, EB)
    dst4d = dstp.reshape(DROWS, DB)

    zseg = jnp.zeros((WB, F), jnp.float32)
    dzeros = jnp.zeros((DSPAN,), jnp.float32)
    dones = jnp.ones((DB,), jnp.float32)

    deg2 = _make_deg()(dst4d, dones, dzeros)[:, :N].T
    st0 = _stats_call(x)
    y_c = _prep_call(x, st0, deg2, gamma0, beta0)
    aggY = _make_seg(PC1)(y_c, src2d, dst2d, zseg)
    Ry = _selfmm_call(y_c, deg2, W1)
    Z1, st1 = _mm1_call(aggY, Ry, deg2, W1, b1)
    h1_c = _bn_call(Z1, st1, gamma1, beta1)
    agg1 = _make_seg(PC2)(h1_c, src2d, dst2d, zseg)
    R2 = _root_call(h1_c, W2_root)
    Z2, st2 = _mm2_call(agg1, R2, W2_rel, b2)
    h2_c = _bn_call(Z2, st2, gamma2, beta2)
    agg2 = _make_seg(PC2)(h2_c, src2d, dst2d, zseg)
    R3 = _root_call(h2_c, W3_root)
    Z3, st3 = _mm2_call(agg2, R3, W3_rel, b3)
    return _bnfinal_call(Z3, st3, gamma3, beta3)


# speedup vs baseline: 1.0133x; 1.0133x over previous
import functools

import jax
import jax.numpy as jnp
from jax import lax
from jax.experimental import pallas as pl
from jax.experimental.pallas import tpu as pltpu
from jax.experimental.pallas import tpu_sc as plsc

N = 10000
E = 160000
C_IN, H1, H2 = 256, 512, 256
EPS = 1e-5

RB = 2000
G = N // RB

F = 64
PC1 = C_IN // F
PC2 = H1 // F
EB = 128
ER = 1280
EPAD = ER * EB
RPT = ER // 16
NPAD = 10240
SPAN = NPAD // 16
WB = 128

DB = 40
DROWS = EPAD // DB
DR = DROWS // 32
DSPAN = NPAD // 16
DPAD = NPAD


@functools.lru_cache(maxsize=None)
def _make_seg(C):
    CPC = C // 2
    mesh = plsc.VectorSubcoreMesh(core_axis_name="c", subcore_axis_name="s",
                                  num_cores=2, num_subcores=16)

    @functools.partial(
        pl.kernel,
        out_type=jax.ShapeDtypeStruct((C, NPAD, F), jnp.float32),
        mesh=mesh,
        compiler_params=pltpu.CompilerParams(use_tc_tiling_on_sc=False),
        scratch_types=[
            pltpu.VMEM((RPT, EB), jnp.int32),
            pltpu.VMEM((RPT, EB), jnp.int32),
            [pltpu.VMEM((EB, F), jnp.float32) for _ in range(4)],
            pltpu.VMEM((WB, F), jnp.float32),
            pltpu.VMEM((WB, F), jnp.float32),
            pltpu.VMEM_SHARED((NPAD, F), jnp.float32),
            [pltpu.SemaphoreType.DMA for _ in range(4)],
            [pltpu.SemaphoreType.DMA for _ in range(4)],
        ],
    )
    def seg(table, src2d, dst2d, zeros_h, out, srcv, dstv, gbufs,
            zbuf, obuf, acc, gsems, ssems):
        c = lax.axis_index("c")
        s = lax.axis_index("s")
        row0 = s * RPT
        pltpu.sync_copy(src2d.at[pl.ds(row0, RPT)], srcv)
        pltpu.sync_copy(dst2d.at[pl.ds(row0, RPT)], dstv)
        pltpu.sync_copy(zeros_h, zbuf)
        for cc in range(CPC):
            chunk = c * CPC + cc
            tch = table.at[chunk]
            och = out.at[chunk]
            for i in range(SPAN // WB):
                pltpu.sync_copy(zbuf, acc.at[pl.ds(s * SPAN + i * WB, WB)])
            plsc.subcore_barrier()
            for b in range(4):
                pltpu.async_copy(tch.at[srcv.at[b]], gbufs[b], gsems[b])

            def quad(p, carry):
                j0 = p * 4
                sdescs = []
                for b in range(4):
                    pltpu.make_async_copy(tch, gbufs[b], gsems[b]).wait()
                    sdescs.append(
                        pltpu.async_copy(gbufs[b], acc.at[dstv.at[j0 + b]],
                                         ssems[b], add=True))
                for b in range(4):
                    sdescs[b].wait()

                    @pl.when(j0 + b + 4 < RPT)
                    def _():
                        pltpu.async_copy(tch.at[srcv.at[j0 + b + 4]],
                                         gbufs[b], gsems[b])
                return carry

            lax.fori_loop(0, RPT // 4, quad, 0)
            plsc.subcore_barrier()
            for i in range(SPAN // WB):
                pltpu.sync_copy(acc.at[pl.ds(s * SPAN + i * WB, WB)], obuf)
                pltpu.sync_copy(obuf, och.at[pl.ds(s * SPAN + i * WB, WB)])
    return seg


@functools.lru_cache(maxsize=None)
def _make_deg():
    mesh = plsc.VectorSubcoreMesh(core_axis_name="c", subcore_axis_name="s",
                                  num_cores=2, num_subcores=16)

    @functools.partial(
        pl.kernel,
        out_type=jax.ShapeDtypeStruct((2, DPAD), jnp.float32),
        mesh=mesh,
        compiler_params=pltpu.CompilerParams(use_tc_tiling_on_sc=False),
        scratch_types=[
            pltpu.VMEM((DR, DB), jnp.int32),
            pltpu.VMEM((DB,), jnp.float32),
            pltpu.VMEM((DSPAN,), jnp.float32),
            pltpu.VMEM((DSPAN,), jnp.float32),
            pltpu.VMEM_SHARED((DPAD,), jnp.float32),
        ],
    )
    def degk(dst4d, ones_h, zeros_h, out, dstv, onesv, zbuf, obuf, acc):
        c = lax.axis_index("c")
        s = lax.axis_index("s")
        row0 = c * (DROWS // 2) + s * DR
        pltpu.sync_copy(dst4d.at[pl.ds(row0, DR)], dstv)
        pltpu.sync_copy(ones_h, onesv)
        pltpu.sync_copy(zeros_h, zbuf)
        pltpu.sync_copy(zbuf, acc.at[pl.ds(s * DSPAN, DSPAN)])
        plsc.subcore_barrier()

        def step(j, carry):
            pltpu.sync_copy(onesv, acc.at[dstv.at[j]], add=True)
            return carry

        lax.fori_loop(0, DR, step, 0)
        plsc.subcore_barrier()
        pltpu.sync_copy(acc.at[pl.ds(s * DSPAN, DSPAN)], obuf)
        pltpu.sync_copy(obuf, out.at[c].at[pl.ds(s * DSPAN, DSPAN)])
    return degk


def _stats_call(x):
    C = x.shape[1]

    def body(x_ref, o_ref):
        i = pl.program_id(0)

        @pl.when(i == 0)
        def _():
            o_ref[...] = jnp.zeros_like(o_ref)

        xb = x_ref[...]
        o_ref[0, :] = o_ref[0, :] + jnp.sum(xb, 0)
        o_ref[1, :] = o_ref[1, :] + jnp.sum(xb * xb, 0)

    return pl.pallas_call(
        body, grid=(G,),
        in_specs=[pl.BlockSpec((RB, C), lambda i: (i, 0))],
        out_specs=pl.BlockSpec((2, C), lambda i: (0, 0)),
        out_shape=jax.ShapeDtypeStruct((2, C), jnp.float32),
    )(x)


def _prep_call(x, st, deg2, g0, b0):
    def body(x_ref, st_ref, dg_ref, g_ref, b_ref, y_ref):
        mean = st_ref[0] / N
        var = st_ref[1] / N - mean * mean
        a = g_ref[...] * lax.rsqrt(var + EPS)
        bb = b_ref[...] - mean * a
        xn = x_ref[...] * a[None, :] + bb[None, :]
        deg = dg_ref[:, 0] + dg_ref[:, 1] + 1.0
        dinv = lax.rsqrt(deg)
        y = xn * dinv[:, None]
        for c in range(PC1):
            y_ref[c] = y[:, c * F:(c + 1) * F]

    return pl.pallas_call(
        body, grid=(G,),
        in_specs=[
            pl.BlockSpec((RB, C_IN), lambda i: (i, 0)),
            pl.BlockSpec((2, C_IN), lambda i: (0, 0)),
            pl.BlockSpec((RB, 2), lambda i: (i, 0)),
            pl.BlockSpec((C_IN,), lambda i: (0,)),
            pl.BlockSpec((C_IN,), lambda i: (0,)),
        ],
        out_specs=pl.BlockSpec((PC1, RB, F), lambda i: (0, i, 0)),
        out_shape=jax.ShapeDtypeStruct((PC1, N, F), jnp.float32),
    )(x, st, deg2, g0, b0)


def _mm1_call(aggS, y_c, deg2, W1, b1):
    def body(ag_ref, y_ref, dg_ref, w_ref, b_ref, z_ref, st_ref):
        i = pl.program_id(0)
        deg = dg_ref[:, 0] + dg_ref[:, 1] + 1.0
        dinv = lax.rsqrt(deg)
        wv = w_ref[...]
        z = jnp.broadcast_to(b_ref[...][None, :], (RB, H1))
        for c in range(PC1):
            m = (ag_ref[c] + y_ref[c]) * dinv[:, None]
            z = z + jnp.dot(m, wv[c * F:(c + 1) * F],
                            preferred_element_type=jnp.float32)
        z = jnp.maximum(z, 0.0)
        z_ref[...] = z

        @pl.when(i == 0)
        def _():
            st_ref[...] = jnp.zeros_like(st_ref)

        st_ref[0, :] = st_ref[0, :] + jnp.sum(z, 0)
        st_ref[1, :] = st_ref[1, :] + jnp.sum(z * z, 0)

    return pl.pallas_call(
        body, grid=(G,),
        in_specs=[
            pl.BlockSpec((PC1, RB, F), lambda i: (0, i, 0)),
            pl.BlockSpec((PC1, RB, F), lambda i: (0, i, 0)),
            pl.BlockSpec((RB, 2), lambda i: (i, 0)),
            pl.BlockSpec((C_IN, H1), lambda i: (0, 0)),
            pl.BlockSpec((H1,), lambda i: (0,)),
        ],
        out_specs=[
            pl.BlockSpec((RB, H1), lambda i: (i, 0)),
            pl.BlockSpec((2, H1), lambda i: (0, 0)),
        ],
        out_shape=[
            jax.ShapeDtypeStruct((N, H1), jnp.float32),
            jax.ShapeDtypeStruct((2, H1), jnp.float32),
        ],
    )(aggS, y_c, deg2, W1, b1)


def _bn_call(Z, st, gamma, beta):
    def body(z_ref, st_ref, g_ref, b_ref, h_ref):
        mean = st_ref[0] / N
        var = st_ref[1] / N - mean * mean
        a = g_ref[...] * lax.rsqrt(var + EPS)
        bb = b_ref[...] - mean * a
        h = z_ref[...] * a[None, :] + bb[None, :]
        for c in range(PC2):
            h_ref[c] = h[:, c * F:(c + 1) * F]

    return pl.pallas_call(
        body, grid=(G,),
        in_specs=[
            pl.BlockSpec((RB, H1), lambda i: (i, 0)),
            pl.BlockSpec((2, H1), lambda i: (0, 0)),
            pl.BlockSpec((H1,), lambda i: (0,)),
            pl.BlockSpec((H1,), lambda i: (0,)),
        ],
        out_specs=pl.BlockSpec((PC2, RB, F), lambda i: (0, i, 0)),
        out_shape=jax.ShapeDtypeStruct((PC2, N, F), jnp.float32),
    )(Z, st, gamma, beta)


def _root_call(h_c, Wroot):
    Hout = Wroot.shape[1]

    def body(h_ref, w_ref, r_ref):
        wv = w_ref[...]
        r = jnp.dot(h_ref[0], wv[:F], preferred_element_type=jnp.float32)
        for c in range(1, PC2):
            r = r + jnp.dot(h_ref[c], wv[c * F:(c + 1) * F],
                            preferred_element_type=jnp.float32)
        r_ref[...] = r

    return pl.pallas_call(
        body, grid=(G,),
        in_specs=[
            pl.BlockSpec((PC2, RB, F), lambda i: (0, i, 0)),
            pl.BlockSpec((H1, Hout), lambda i: (0, 0)),
        ],
        out_specs=pl.BlockSpec((RB, Hout), lambda i: (i, 0)),
        out_shape=jax.ShapeDtypeStruct((N, Hout), jnp.float32),
    )(h_c, Wroot)


def _mm2_call(aggS, R, W_rel, b):
    Hout = W_rel.shape[1]

    def body(ag_ref, r_ref, w_ref, b_ref, z_ref, st_ref):
        i = pl.program_id(0)
        wv = w_ref[...]
        z = r_ref[...] + b_ref[...][None, :]
        for c in range(PC2):
            z = z + jnp.dot(ag_ref[c], wv[c * F:(c + 1) * F],
                            preferred_element_type=jnp.float32)
        z = jnp.maximum(z, 0.0)
        z_ref[...] = z

        @pl.when(i == 0)
        def _():
            st_ref[...] = jnp.zeros_like(st_ref)

        st_ref[0, :] = st_ref[0, :] + jnp.sum(z, 0)
        st_ref[1, :] = st_ref[1, :] + jnp.sum(z * z, 0)

    return pl.pallas_call(
        body, grid=(G,),
        in_specs=[
            pl.BlockSpec((PC2, RB, F), lambda i: (0, i, 0)),
            pl.BlockSpec((RB, Hout), lambda i: (i, 0)),
            pl.BlockSpec((H1, Hout), lambda i: (0, 0)),
            pl.BlockSpec((Hout,), lambda i: (0,)),
        ],
        out_specs=[
            pl.BlockSpec((RB, Hout), lambda i: (i, 0)),
            pl.BlockSpec((2, Hout), lambda i: (0, 0)),
        ],
        out_shape=[
            jax.ShapeDtypeStruct((N, Hout), jnp.float32),
            jax.ShapeDtypeStruct((2, Hout), jnp.float32),
        ],
    )(aggS, R, W_rel, b)


def _bnfinal_call(Z, st, gamma, beta):
    C = Z.shape[1]

    def body(z_ref, st_ref, g_ref, b_ref, o_ref):
        mean = st_ref[0] / N
        var = st_ref[1] / N - mean * mean
        a = g_ref[...] * lax.rsqrt(var + EPS)
        bb = b_ref[...] - mean * a
        o_ref[...] = z_ref[...] * a[None, :] + bb[None, :]

    return pl.pallas_call(
        body, grid=(G,),
        in_specs=[
            pl.BlockSpec((RB, C), lambda i: (i, 0)),
            pl.BlockSpec((2, C), lambda i: (0, 0)),
            pl.BlockSpec((C,), lambda i: (0,)),
            pl.BlockSpec((C,), lambda i: (0,)),
        ],
        out_specs=pl.BlockSpec((RB, C), lambda i: (i, 0)),
        out_shape=jax.ShapeDtypeStruct((N, C), jnp.float32),
    )(Z, st, gamma, beta)


def kernel(x, edge_index, gamma0, beta0, W1, b1, gamma1, beta1,
           W2_rel, b2, W2_root, gamma2, beta2,
           W3_rel, b3, W3_root, gamma3, beta3):
    pad = jnp.arange(EPAD - E, dtype=jnp.int32)
    srcp = jnp.concatenate([edge_index[0], pad % N])
    dstp = jnp.concatenate([edge_index[1], N + pad % (NPAD - N)])
    src2d = srcp.reshape(ER, EB)
    dst2d = dstp.reshape(ER, EB)
    dst4d = dstp.reshape(DROWS, DB)

    zseg = jnp.zeros((WB, F), jnp.float32)
    dzeros = jnp.zeros((DSPAN,), jnp.float32)
    dones = jnp.ones((DB,), jnp.float32)

    deg2 = _make_deg()(dst4d, dones, dzeros)[:, :N].T
    st0 = _stats_call(x)
    y_c = _prep_call(x, st0, deg2, gamma0, beta0)
    aggY = _make_seg(PC1)(y_c, src2d, dst2d, zseg)
    Z1, st1 = _mm1_call(aggY, y_c, deg2, W1, b1)
    h1_c = _bn_call(Z1, st1, gamma1, beta1)
    agg1 = _make_seg(PC2)(h1_c, src2d, dst2d, zseg)
    R2 = _root_call(h1_c, W2_root)
    Z2, st2 = _mm2_call(agg1, R2, W2_rel, b2)
    h2_c = _bn_call(Z2, st2, gamma2, beta2)
    agg2 = _make_seg(PC2)(h2_c, src2d, dst2d, zseg)
    R3 = _root_call(h2_c, W3_root)
    Z3, st3 = _mm2_call(agg2, R3, W3_rel, b3)
    return _bnfinal_call(Z3, st3, gamma3, beta3)
